# baseline (speedup 1.0000x reference)
import functools

import jax
import jax.numpy as jnp
from jax import lax
from jax.experimental import pallas as pl
from jax.experimental.pallas import tpu as pltpu
from jax.experimental.pallas import tpu_sc as plsc

B = 16384
F = 26
V = 1_040_000

NC = 2
NS = 16
NW = NC * NS
BPW = B // NW
NIDX = BPW * F
VS = V // NS
SCH = 13000


def _sc_lookup_sum(xT, tableT, bias):
    mesh = plsc.VectorSubcoreMesh(core_axis_name="c", subcore_axis_name="s")

    @functools.partial(
        pl.kernel,
        mesh=mesh,
        compiler_params=pltpu.CompilerParams(
            needs_layout_passes=False,
            disable_bounds_checks=True,
            disable_semaphore_checks=True,
            skip_device_barrier=True,
        ),
        out_type=jax.ShapeDtypeStruct((1, B), jnp.float32),
        scratch_types=[
            pltpu.VMEM((NIDX,), jnp.int32),
            pltpu.VMEM((NIDX,), jnp.float32),
            pltpu.VMEM((BPW,), jnp.float32),
            pltpu.VMEM((16,), jnp.float32),
            pltpu.VMEM((SCH,), jnp.float32),
            pltpu.VMEM((SCH,), jnp.float32),
            pltpu.VMEM_SHARED((V,), jnp.float32),
            pltpu.SemaphoreType.DMA,
            pltpu.SemaphoreType.DMA,
            pltpu.SemaphoreType.DMA,
        ],
    )
    def k(xT_hbm, table_hbm, bias_hbm, out_hbm, idx_v, vals_v, out_v, bias_v,
          stage_a, stage_b, table_sh, sem, sem2, sem3):
        sid = lax.axis_index("s")
        wid = sid * NC + lax.axis_index("c")
        base = wid * BPW

        zidx = lax.iota(jnp.int32, 16) * 0
        bias_cp = pltpu.async_copy(bias_hbm.at[zidx], bias_v, sem3)
        idx_cps = [
            pltpu.async_copy(
                xT_hbm.at[f, pl.ds(base, BPW)],
                idx_v.at[pl.ds(f * BPW, BPW)], sem3,
            )
            for f in range(F)
        ]

        stages = [stage_a, stage_b]
        out_cps = [None, None]
        for j in range(VS // SCH):
            off = sid * VS + j * SCH
            if out_cps[j % 2] is not None:
                out_cps[j % 2].wait()
            pltpu.async_copy(
                table_hbm.at[0].at[pl.ds(off, SCH)], stages[j % 2], sem
            ).wait()
            out_cps[j % 2] = pltpu.async_copy(
                stages[j % 2], table_sh.at[pl.ds(off, SCH)], sem2
            )
        for cp in out_cps:
            cp.wait()
        for cp in idx_cps:
            cp.wait()
        plsc.subcore_barrier()

        HF = F // 2
        g1 = pltpu.async_copy(
            table_sh.at[idx_v.at[pl.ds(0, HF * BPW)]],
            vals_v.at[pl.ds(0, HF * BPW)], sem,
        )
        g2 = pltpu.async_copy(
            table_sh.at[idx_v.at[pl.ds(HF * BPW, (F - HF) * BPW)]],
            vals_v.at[pl.ds(HF * BPW, (F - HF) * BPW)], sem2,
        )
        bias_cp.wait()
        g1.wait()
        bv = bias_v[...]
        accs = []
        for g in range(BPW // 16):
            acc = bv
            for f in range(HF):
                acc = acc + vals_v[pl.ds(f * BPW + g * 16, 16)]
            accs.append(acc)
        g2.wait()
        for g in range(BPW // 16):
            acc = accs[g]
            for f in range(HF, F):
                acc = acc + vals_v[pl.ds(f * BPW + g * 16, 16)]
            out_v[pl.ds(g * 16, 16)] = acc

        pltpu.sync_copy(out_v, out_hbm.at[0].at[pl.ds(base, BPW)])

    return k(xT, tableT, bias)


def kernel(x, fc_weight, bias):
    out = _sc_lookup_sum(x.T, fc_weight.T, bias.astype(jnp.float32))
    return out.T


# speedup vs baseline: 1.0918x; 1.0918x over previous
import functools

import jax
import jax.numpy as jnp
from jax import lax
from jax.experimental import pallas as pl
from jax.experimental.pallas import tpu as pltpu
from jax.experimental.pallas import tpu_sc as plsc

B = 16384
F = 26
V = 1_040_000

NC = 2
NS = 16
NW = NC * NS
BPW = B // NW
NIDX = BPW * F
VS = V // NS
SCH = 13000


def _sc_lookup_sum(xT, tableT, bias):
    mesh = plsc.VectorSubcoreMesh(core_axis_name="c", subcore_axis_name="s")

    @functools.partial(
        pl.kernel,
        mesh=mesh,
        compiler_params=pltpu.CompilerParams(
            needs_layout_passes=False,
            disable_bounds_checks=True,
            disable_semaphore_checks=True,
            skip_device_barrier=True,
        ),
        out_type=jax.ShapeDtypeStruct((1, B), jnp.float32),
        scratch_types=[
            pltpu.VMEM((NIDX,), jnp.int32),
            pltpu.VMEM((NIDX,), jnp.float32),
            pltpu.VMEM((BPW,), jnp.float32),
            pltpu.VMEM((16,), jnp.float32),
            pltpu.VMEM((SCH,), jnp.float32),
            pltpu.VMEM((SCH,), jnp.float32),
            pltpu.VMEM_SHARED((V,), jnp.float32),
            pltpu.SemaphoreType.DMA,
            pltpu.SemaphoreType.DMA,
            pltpu.SemaphoreType.DMA,
        ],
    )
    def k(xT_hbm, table_hbm, bias_hbm, out_hbm, idx_v, vals_v, out_v, bias_v,
          stage_a, stage_b, table_sh, sem, sem2, sem3):
        sid = lax.axis_index("s")
        wid = sid * NC + lax.axis_index("c")
        base = wid * BPW

        zidx = lax.iota(jnp.int32, 16) * 0
        bias_cp = pltpu.async_copy(bias_hbm.at[zidx], bias_v, sem3)
        idx_cps = [
            pltpu.async_copy(
                xT_hbm.at[f, pl.ds(base, BPW)],
                idx_v.at[pl.ds(f * BPW, BPW)], sem3,
            )
            for f in range(F)
        ]

        stages = [stage_a, stage_b]
        out_cps = [None, None]
        for j in range(VS // SCH):
            off = sid * VS + j * SCH
            if out_cps[j % 2] is not None:
                out_cps[j % 2].wait()
            pltpu.async_copy(
                table_hbm.at[0].at[pl.ds(off, SCH)], stages[j % 2], sem
            ).wait()
            out_cps[j % 2] = pltpu.async_copy(
                stages[j % 2], table_sh.at[pl.ds(off, SCH)], sem2
            )
        for cp in out_cps:
            cp.wait()
        for cp in idx_cps:
            cp.wait()
        plsc.subcore_barrier()

        HF = F // 2
        g1 = pltpu.async_copy(
            table_sh.at[idx_v.at[pl.ds(0, HF * BPW)]],
            vals_v.at[pl.ds(0, HF * BPW)], sem,
        )
        g2 = pltpu.async_copy(
            table_sh.at[idx_v.at[pl.ds(HF * BPW, (F - HF) * BPW)]],
            vals_v.at[pl.ds(HF * BPW, (F - HF) * BPW)], sem2,
        )
        bias_cp.wait()
        g1.wait()
        bv = bias_v[...]

        def acc_lo(g, _):
            o = g * 16
            acc = bv
            for f in range(HF):
                acc = acc + vals_v[pl.ds(f * BPW + o, 16)]
            out_v[pl.ds(o, 16)] = acc
            return _

        lax.fori_loop(0, BPW // 16, acc_lo, None, unroll=False)
        g2.wait()

        def acc_hi(g, _):
            o = g * 16
            acc = out_v[pl.ds(o, 16)]
            for f in range(HF, F):
                acc = acc + vals_v[pl.ds(f * BPW + o, 16)]
            out_v[pl.ds(o, 16)] = acc
            return _

        lax.fori_loop(0, BPW // 16, acc_hi, None, unroll=False)

        pltpu.sync_copy(out_v, out_hbm.at[0].at[pl.ds(base, BPW)])

    return k(xT, tableT, bias)


def kernel(x, fc_weight, bias):
    out = _sc_lookup_sum(x.T, fc_weight.T, bias.astype(jnp.float32))
    return out.T
